# initial kernel scaffold (unmeasured)
import jax
import jax.numpy as jnp
from jax import lax
from jax.experimental import pallas as pl
from jax.experimental.pallas import tpu as pltpu

N_DEV = 4


def kernel(A, B):
    A = A.astype(jnp.bfloat16)
    B = B.astype(jnp.bfloat16)
    m_per, k = A.shape
    n = B.shape[1]
    M = N_DEV * m_per

    def body(a_ref, b_ref, out_ref, comm_ref, acc_ref, send_sems, recv_sems, copy_sem):
        my = lax.axis_index("i")
        left = (my - 1) % N_DEV
        right = (my + 1) % N_DEV

        barrier_sem = pltpu.get_barrier_semaphore()
        for nbr in (left, right):
            pl.semaphore_signal(
                barrier_sem, inc=1,
                device_id=(nbr,), device_id_type=pl.DeviceIdType.MESH,
            )
        pl.semaphore_wait(barrier_sem, 2)

        comm_ref[0] = a_ref[...]
        acc_ref[...] = jnp.dot(
            a_ref[...], b_ref[...], preferred_element_type=jnp.float32
        ).astype(jnp.bfloat16)
        copy = pltpu.make_async_copy(
            acc_ref, out_ref.at[pl.ds(my * m_per, m_per)], copy_sem
        )
        copy.start()
        copy.wait()

        for h in range(N_DEV - 1):
            send_slot = h % 2
            recv_slot = (h + 1) % 2
            rdma = pltpu.make_async_remote_copy(
                src_ref=comm_ref.at[send_slot],
                dst_ref=comm_ref.at[recv_slot],
                send_sem=send_sems.at[send_slot],
                recv_sem=recv_sems.at[recv_slot],
                device_id=(right,),
                device_id_type=pl.DeviceIdType.MESH,
            )
            rdma.start()
            rdma.wait()

            origin = (my - h - 1) % N_DEV
            acc_ref[...] = jnp.dot(
                comm_ref[recv_slot], b_ref[...], preferred_element_type=jnp.float32
            ).astype(jnp.bfloat16)
            copy = pltpu.make_async_copy(
                acc_ref, out_ref.at[pl.ds(origin * m_per, m_per)], copy_sem
            )
            copy.start()
            copy.wait()

    return pl.pallas_call(
        body,
        out_shape=jax.ShapeDtypeStruct((M, n), jnp.bfloat16),
        in_specs=[
            pl.BlockSpec(memory_space=pltpu.VMEM),
            pl.BlockSpec(memory_space=pltpu.VMEM),
        ],
        out_specs=pl.BlockSpec(memory_space=pltpu.ANY),
        scratch_shapes=[
            pltpu.VMEM((2, m_per, k), jnp.bfloat16),
            pltpu.VMEM((m_per, n), jnp.bfloat16),
            pltpu.SemaphoreType.DMA((2,)),
            pltpu.SemaphoreType.DMA((2,)),
            pltpu.SemaphoreType.DMA,
        ],
        compiler_params=pltpu.CompilerParams(collective_id=0),
    )(A, B)


# baseline (device time: 423554 ns/iter reference)
import jax
import jax.numpy as jnp
from jax import lax
from jax.experimental import pallas as pl
from jax.experimental.pallas import tpu as pltpu

N_DEV = 4
N_SLOT = 3
M_SPLIT = 2


def kernel(A, B):
    A = A.astype(jnp.bfloat16)
    B = B.astype(jnp.bfloat16)
    m_per, k = A.shape
    n = B.shape[1]
    M = N_DEV * m_per
    m_half = m_per // M_SPLIT

    def body(a_ref, b_ref, out_ref, comm_ref, acc_ref, send_sems, recv_sems,
             credit_sem, in_sem, out_sem):
        my = lax.axis_index("i")
        left = (my - 1) % N_DEV
        right = (my + 1) % N_DEV

        in_copy = pltpu.make_async_copy(a_ref, comm_ref.at[0], in_sem)
        in_copy.start()

        barrier_sem = pltpu.get_barrier_semaphore()
        for nbr in (left, right):
            pl.semaphore_signal(
                barrier_sem, inc=1,
                device_id=(nbr,), device_id_type=pl.DeviceIdType.MESH,
            )
        pl.semaphore_wait(barrier_sem, 2)
        in_copy.wait()

        def step(h, carry):
            slot = lax.rem(h, N_SLOT)
            dst_slot = lax.rem(h + 1, N_SLOT)

            @pl.when(h < N_DEV - 1)
            def _start():
                @pl.when(h == 2)
                def _():
                    pl.semaphore_wait(credit_sem, 1)
                rdma = pltpu.make_async_remote_copy(
                    src_ref=comm_ref.at[slot],
                    dst_ref=comm_ref.at[dst_slot],
                    send_sem=send_sems.at[h],
                    recv_sem=recv_sems.at[h],
                    device_id=(right,),
                    device_id_type=pl.DeviceIdType.MESH,
                )
                rdma.start()

            origin = lax.rem(my - h + N_DEV, N_DEV)

            def half(t, c):
                acc_ref[...] = jnp.dot(
                    comm_ref[slot, pl.ds(t * m_half, m_half), :],
                    b_ref[...],
                    preferred_element_type=jnp.float32,
                ).astype(jnp.bfloat16)
                out_copy = pltpu.make_async_copy(
                    acc_ref,
                    out_ref.at[pl.ds(origin * m_per + t * m_half, m_half)],
                    out_sem,
                )
                out_copy.start()
                out_copy.wait()
                return c

            lax.fori_loop(0, M_SPLIT, half, 0, unroll=False)

            @pl.when(h < N_DEV - 1)
            def _wait():
                rdma = pltpu.make_async_remote_copy(
                    src_ref=comm_ref.at[slot],
                    dst_ref=comm_ref.at[dst_slot],
                    send_sem=send_sems.at[h],
                    recv_sem=recv_sems.at[h],
                    device_id=(right,),
                    device_id_type=pl.DeviceIdType.MESH,
                )
                rdma.wait()

            @pl.when(h == 0)
            def _credit():
                pl.semaphore_signal(
                    credit_sem, inc=1,
                    device_id=(left,), device_id_type=pl.DeviceIdType.MESH,
                )
            return carry

        lax.fori_loop(0, N_DEV, step, 0, unroll=False)

    return pl.pallas_call(
        body,
        out_shape=jax.ShapeDtypeStruct((M, n), jnp.bfloat16),
        in_specs=[
            pl.BlockSpec(memory_space=pl.ANY),
            pl.BlockSpec(memory_space=pltpu.VMEM),
        ],
        out_specs=pl.BlockSpec(memory_space=pl.ANY),
        scratch_shapes=[
            pltpu.VMEM((N_SLOT, m_per, k), jnp.bfloat16),
            pltpu.VMEM((m_half, n), jnp.bfloat16),
            pltpu.SemaphoreType.DMA((N_DEV - 1,)),
            pltpu.SemaphoreType.DMA((N_DEV - 1,)),
            pltpu.SemaphoreType.REGULAR,
            pltpu.SemaphoreType.DMA,
            pltpu.SemaphoreType.DMA,
        ],
        compiler_params=pltpu.CompilerParams(
            collective_id=0,
            vmem_limit_bytes=60 * 1024 * 1024,
        ),
    )(A, B)


# device time: 401637 ns/iter; 1.0546x vs baseline; 1.0546x over previous
import jax
import jax.numpy as jnp
from jax import lax
from jax.experimental import pallas as pl
from jax.experimental.pallas import tpu as pltpu

N_DEV = 4
N_SLOT = 3
N_SEG = 2
N_HOP = N_DEV - 1
N_SUB = N_HOP * N_SEG


def kernel(A, B):
    A = A.astype(jnp.bfloat16)
    B = B.astype(jnp.bfloat16)
    m_per, k = A.shape
    n = B.shape[1]
    M = N_DEV * m_per
    m_half = m_per // N_SEG

    def body(a_ref, b_ref, out_ref, comm_ref, acc_ref, send_sems, recv_sems,
             credit_sem, in_sem, out_sem):
        my = lax.axis_index("i")
        left = (my - 1) % N_DEV
        right = (my + 1) % N_DEV

        in_copy = pltpu.make_async_copy(a_ref, comm_ref.at[0], in_sem)
        in_copy.start()

        barrier_sem = pltpu.get_barrier_semaphore()
        for nbr in (left, right):
            pl.semaphore_signal(
                barrier_sem, inc=1,
                device_id=(nbr,), device_id_type=pl.DeviceIdType.MESH,
            )
        pl.semaphore_wait(barrier_sem, 2)
        in_copy.wait()

        def sub_rdma(g):
            h = lax.div(g, N_SEG)
            s = lax.rem(g, N_SEG)
            rows = pl.ds(s * m_half, m_half)
            return pltpu.make_async_remote_copy(
                src_ref=comm_ref.at[lax.rem(h, N_SLOT), rows],
                dst_ref=comm_ref.at[lax.rem(h + 1, N_SLOT), rows],
                send_sem=send_sems.at[g],
                recv_sem=recv_sems.at[g],
                device_id=(right,),
                device_id_type=pl.DeviceIdType.MESH,
            )

        def step(g, carry):
            h = lax.div(g, N_SEG)
            s = lax.rem(g, N_SEG)
            slot = lax.rem(h, N_SLOT)

            @pl.when(h > 0)
            def _recv():
                sub_rdma((h - 1) * N_SEG + s).wait_recv()

            @pl.when(h < N_HOP)
            def _send():
                @pl.when(jnp.logical_and(h == N_HOP - 1, s == 0))
                def _():
                    pl.semaphore_wait(credit_sem, 1)
                sub_rdma(g).start()

            origin = lax.rem(my - h + N_DEV, N_DEV)
            acc_ref[...] = jnp.dot(
                comm_ref[slot, pl.ds(s * m_half, m_half), :],
                b_ref[...],
                preferred_element_type=jnp.float32,
            ).astype(jnp.bfloat16)
            out_copy = pltpu.make_async_copy(
                acc_ref,
                out_ref.at[pl.ds(origin * m_per + s * m_half, m_half)],
                out_sem,
            )
            out_copy.start()
            out_copy.wait()

            @pl.when(g == 1)
            def _credit():
                sub_rdma(0).wait_send()
                sub_rdma(1).wait_send()
                pl.semaphore_signal(
                    credit_sem, inc=1,
                    device_id=(left,), device_id_type=pl.DeviceIdType.MESH,
                )
            return carry

        lax.fori_loop(0, N_DEV * N_SEG, step, 0, unroll=False)

        def drain(g, carry):
            sub_rdma(g).wait_send()
            return carry

        lax.fori_loop(2, N_SUB, drain, 0, unroll=False)

    return pl.pallas_call(
        body,
        out_shape=jax.ShapeDtypeStruct((M, n), jnp.bfloat16),
        in_specs=[
            pl.BlockSpec(memory_space=pl.ANY),
            pl.BlockSpec(memory_space=pltpu.VMEM),
        ],
        out_specs=pl.BlockSpec(memory_space=pl.ANY),
        scratch_shapes=[
            pltpu.VMEM((N_SLOT, m_per, k), jnp.bfloat16),
            pltpu.VMEM((m_half, n), jnp.bfloat16),
            pltpu.SemaphoreType.DMA((N_SUB,)),
            pltpu.SemaphoreType.DMA((N_SUB,)),
            pltpu.SemaphoreType.REGULAR,
            pltpu.SemaphoreType.DMA,
            pltpu.SemaphoreType.DMA,
        ],
        compiler_params=pltpu.CompilerParams(
            collective_id=0,
            vmem_limit_bytes=60 * 1024 * 1024,
        ),
    )(A, B)
